# baseline (speedup 1.0000x reference)
import functools

import jax
import jax.numpy as jnp
from jax import lax
from jax.experimental import pallas as pl
from jax.experimental.pallas import tpu as pltpu

_BN_EPS = 1e-5
_VMEM_LIMIT = 32 * 1024 * 1024


def _patches3(xb):
    z = jnp.zeros((xb.shape[0], 1), jnp.bfloat16)
    xm = jnp.concatenate([z, xb[:, :-1]], axis=1)
    xp = jnp.concatenate([xb[:, 1:], z], axis=1)
    return jnp.concatenate([xm, xb, xp], axis=0)


def _stats_kernel(n_total, x_ref, w_ref, g_ref, b_ref,
                  xb_ref, ss_ref, acc_ref):
    i = pl.program_id(0)

    @pl.when(i == 0)
    def _():
        acc_ref[...] = jnp.zeros_like(acc_ref)

    xb = x_ref[0].astype(jnp.bfloat16)
    xb_ref[0] = xb
    p = _patches3(xb)
    y = jnp.dot(w_ref[...], p, preferred_element_type=jnp.float32)
    s = jnp.sum(y, axis=1, keepdims=True)
    ss = jnp.sum(y * y, axis=1, keepdims=True)
    acc_ref[...] += jnp.concatenate([s, ss], axis=1)

    @pl.when(i == n_total - 1)
    def _():
        inv_m = 1.0 / (n_total * y.shape[1])
        mean = acc_ref[:, 0:1] * inv_m
        var = jnp.maximum(acc_ref[:, 1:2] * inv_m - mean * mean, 0.0)
        scale = g_ref[...] * lax.rsqrt(var + _BN_EPS)
        shift = b_ref[...] - mean * scale
        ss_ref[...] = jnp.concatenate([scale, shift], axis=1)


def _apply_kernel(xb_ref, w_ref, ss_ref, o_ref, ws_ref):
    @pl.when(pl.program_id(0) == 0)
    def _():
        ws_ref[...] = (w_ref[...] * ss_ref[:, 0:1]).astype(jnp.bfloat16)

    p = _patches3(xb_ref[0])
    y = jnp.dot(ws_ref[...], p, preferred_element_type=jnp.float32)
    o_ref[0] = jnp.maximum(y + ss_ref[:, 1:2], 0.0)


def kernel(x, conv_b, conv_w, bn_gamma, bn_beta):
    del conv_b
    n, c_in, l = x.shape
    c_out = conv_w.shape[0]
    ck = 3 * c_in
    w_flat = conv_w.transpose(0, 2, 1).reshape(c_out, ck)

    params_seq = pltpu.CompilerParams(
        dimension_semantics=("arbitrary",), vmem_limit_bytes=_VMEM_LIMIT)

    xb, ss = pl.pallas_call(
        functools.partial(_stats_kernel, n),
        out_shape=(
            jax.ShapeDtypeStruct((n, c_in, l), jnp.bfloat16),
            jax.ShapeDtypeStruct((c_out, 2), jnp.float32),
        ),
        grid=(n,),
        in_specs=[
            pl.BlockSpec((1, c_in, l), lambda i: (i, 0, 0)),
            pl.BlockSpec((c_out, ck), lambda i: (0, 0)),
            pl.BlockSpec((c_out, 1), lambda i: (0, 0)),
            pl.BlockSpec((c_out, 1), lambda i: (0, 0)),
        ],
        out_specs=(
            pl.BlockSpec((1, c_in, l), lambda i: (i, 0, 0)),
            pl.BlockSpec((c_out, 2), lambda i: (0, 0)),
        ),
        scratch_shapes=[pltpu.VMEM((c_out, 2), jnp.float32)],
        compiler_params=params_seq,
    )(x, w_flat.astype(jnp.bfloat16), bn_gamma.reshape(c_out, 1),
      bn_beta.reshape(c_out, 1))

    return pl.pallas_call(
        _apply_kernel,
        out_shape=jax.ShapeDtypeStruct((n, c_out, l), jnp.float32),
        grid=(n,),
        in_specs=[
            pl.BlockSpec((1, c_in, l), lambda i: (i, 0, 0)),
            pl.BlockSpec((c_out, ck), lambda i: (0, 0)),
            pl.BlockSpec((c_out, 2), lambda i: (0, 0)),
        ],
        out_specs=pl.BlockSpec((1, c_out, l), lambda i: (i, 0, 0)),
        scratch_shapes=[pltpu.VMEM((c_out, ck), jnp.bfloat16)],
        compiler_params=params_seq,
    )(xb, w_flat, ss)


# speedup vs baseline: 1.2107x; 1.2107x over previous
import jax
import jax.numpy as jnp
from jax import lax
from jax.experimental import pallas as pl
from jax.experimental.pallas import tpu as pltpu

_BN_EPS = 1e-5
_VMEM_LIMIT = 32 * 1024 * 1024
_PASS1_BLOCK = 2


def _patches3(xb):
    z = jnp.zeros((xb.shape[0], 1), jnp.bfloat16)
    xm = jnp.concatenate([z, xb[:, :-1]], axis=1)
    xp = jnp.concatenate([xb[:, 1:], z], axis=1)
    return jnp.concatenate([xm, xb, xp], axis=0)


def _stats_kernel(x_ref, w_ref, xb_ref, stats_ref):
    i = pl.program_id(0)

    @pl.when(i == 0)
    def _():
        stats_ref[...] = jnp.zeros_like(stats_ref)

    acc = None
    for k in range(x_ref.shape[0]):
        xb = x_ref[k].astype(jnp.bfloat16)
        xb_ref[k] = xb
        p = _patches3(xb)
        y = jnp.dot(w_ref[...], p, preferred_element_type=jnp.float32)
        s = jnp.sum(y, axis=1, keepdims=True)
        ss = jnp.sum(y * y, axis=1, keepdims=True)
        c = jnp.concatenate([s, ss], axis=1)
        acc = c if acc is None else acc + c
    stats_ref[...] += acc


def _apply_kernel(xb_ref, w_ref, shift_ref, o_ref):
    p = _patches3(xb_ref[0])
    y = jnp.dot(w_ref[...], p, preferred_element_type=jnp.float32)
    o_ref[0] = jnp.maximum(y + shift_ref[...], 0.0)


def kernel(x, conv_b, conv_w, bn_gamma, bn_beta):
    del conv_b
    n, c_in, l = x.shape
    c_out = conv_w.shape[0]
    m = n * l
    ck = 3 * c_in
    nb = _PASS1_BLOCK
    w_flat = conv_w.transpose(0, 2, 1).reshape(c_out, ck)

    params_seq = pltpu.CompilerParams(
        dimension_semantics=("arbitrary",), vmem_limit_bytes=_VMEM_LIMIT)

    xb, tot = pl.pallas_call(
        _stats_kernel,
        out_shape=(
            jax.ShapeDtypeStruct((n, c_in, l), jnp.bfloat16),
            jax.ShapeDtypeStruct((c_out, 2), jnp.float32),
        ),
        grid=(n // nb,),
        in_specs=[
            pl.BlockSpec((nb, c_in, l), lambda i: (i, 0, 0)),
            pl.BlockSpec((c_out, ck), lambda i: (0, 0)),
        ],
        out_specs=(
            pl.BlockSpec((nb, c_in, l), lambda i: (i, 0, 0)),
            pl.BlockSpec((c_out, 2), lambda i: (0, 0)),
        ),
        compiler_params=params_seq,
    )(x, w_flat.astype(jnp.bfloat16))

    mean = tot[:, 0] / m
    var = jnp.maximum(tot[:, 1] / m - mean * mean, 0.0)
    scale = bn_gamma * lax.rsqrt(var + _BN_EPS)
    shift = (bn_beta - mean * scale).reshape(c_out, 1)
    w_scaled = (w_flat * scale[:, None]).astype(jnp.bfloat16)

    return pl.pallas_call(
        _apply_kernel,
        out_shape=jax.ShapeDtypeStruct((n, c_out, l), jnp.float32),
        grid=(n,),
        in_specs=[
            pl.BlockSpec((1, c_in, l), lambda i: (i, 0, 0)),
            pl.BlockSpec((c_out, ck), lambda i: (0, 0)),
            pl.BlockSpec((c_out, 1), lambda i: (0, 0)),
        ],
        out_specs=pl.BlockSpec((1, c_out, l), lambda i: (i, 0, 0)),
        compiler_params=params_seq,
    )(xb, w_scaled, shift)


# speedup vs baseline: 1.3353x; 1.1028x over previous
import jax
import jax.numpy as jnp
from jax import lax
from jax.experimental import pallas as pl
from jax.experimental.pallas import tpu as pltpu

_BN_EPS = 1e-5
_VMEM_LIMIT = 32 * 1024 * 1024
_PASS1_BLOCK = 4


def _patches3(xb):
    z = jnp.zeros((xb.shape[0], 1), jnp.bfloat16)
    xm = jnp.concatenate([z, xb[:, :-1]], axis=1)
    xp = jnp.concatenate([xb[:, 1:], z], axis=1)
    return jnp.concatenate([xm, xb, xp], axis=0)


def _stats_kernel(x_ref, w_ref, xb_ref, stats_ref):
    i = pl.program_id(0)

    @pl.when(i == 0)
    def _():
        stats_ref[...] = jnp.zeros_like(stats_ref)

    acc = None
    for k in range(x_ref.shape[0]):
        xb = x_ref[k].astype(jnp.bfloat16)
        xb_ref[k] = xb
        p = _patches3(xb)
        y = jnp.dot(w_ref[...], p, preferred_element_type=jnp.float32)
        s = jnp.sum(y, axis=1, keepdims=True)
        ss = jnp.sum(y * y, axis=1, keepdims=True)
        c = jnp.concatenate([s, ss], axis=1)
        acc = c if acc is None else acc + c
    stats_ref[...] += acc


def _apply_kernel(xb_ref, w_ref, shift_ref, o_ref):
    p = _patches3(xb_ref[0])
    y = jnp.dot(w_ref[...], p, preferred_element_type=jnp.float32)
    o_ref[0] = jnp.maximum(y + shift_ref[...], 0.0)


def kernel(x, conv_b, conv_w, bn_gamma, bn_beta):
    del conv_b
    n, c_in, l = x.shape
    c_out = conv_w.shape[0]
    m = n * l
    ck = 3 * c_in
    nb = _PASS1_BLOCK
    w_flat = conv_w.transpose(0, 2, 1).reshape(c_out, ck)

    params_seq = pltpu.CompilerParams(
        dimension_semantics=("arbitrary",), vmem_limit_bytes=_VMEM_LIMIT)

    xb, tot = pl.pallas_call(
        _stats_kernel,
        out_shape=(
            jax.ShapeDtypeStruct((n, c_in, l), jnp.bfloat16),
            jax.ShapeDtypeStruct((c_out, 2), jnp.float32),
        ),
        grid=(n // nb,),
        in_specs=[
            pl.BlockSpec((nb, c_in, l), lambda i: (i, 0, 0)),
            pl.BlockSpec((c_out, ck), lambda i: (0, 0)),
        ],
        out_specs=(
            pl.BlockSpec((nb, c_in, l), lambda i: (i, 0, 0)),
            pl.BlockSpec((c_out, 2), lambda i: (0, 0)),
        ),
        compiler_params=params_seq,
    )(x, w_flat.astype(jnp.bfloat16))

    mean = tot[:, 0] / m
    var = jnp.maximum(tot[:, 1] / m - mean * mean, 0.0)
    scale = bn_gamma * lax.rsqrt(var + _BN_EPS)
    shift = (bn_beta - mean * scale).reshape(c_out, 1)
    w_scaled = (w_flat * scale[:, None]).astype(jnp.bfloat16)

    return pl.pallas_call(
        _apply_kernel,
        out_shape=jax.ShapeDtypeStruct((n, c_out, l), jnp.float32),
        grid=(n,),
        in_specs=[
            pl.BlockSpec((1, c_in, l), lambda i: (i, 0, 0)),
            pl.BlockSpec((c_out, ck), lambda i: (0, 0)),
            pl.BlockSpec((c_out, 1), lambda i: (0, 0)),
        ],
        out_specs=pl.BlockSpec((1, c_out, l), lambda i: (i, 0, 0)),
        compiler_params=params_seq,
    )(xb, w_scaled, shift)


# speedup vs baseline: 1.6275x; 1.2189x over previous
import jax
import jax.numpy as jnp
from jax import lax
from jax.experimental import pallas as pl
from jax.experimental.pallas import tpu as pltpu

_BN_EPS = 1e-5
_VMEM_LIMIT = 32 * 1024 * 1024
_PASS1_BLOCK = 8
_PASS2_BLOCK = 2


def _patches3(xb):
    z = jnp.zeros((xb.shape[0], 1), jnp.bfloat16)
    xm = jnp.concatenate([z, xb[:, :-1]], axis=1)
    xp = jnp.concatenate([xb[:, 1:], z], axis=1)
    return jnp.concatenate([xm, xb, xp], axis=0)


def _stats_kernel(x_ref, w_ref, xb_ref, stats_ref):
    i = pl.program_id(0)

    @pl.when(i == 0)
    def _():
        stats_ref[...] = jnp.zeros_like(stats_ref)

    acc = None
    for k in range(x_ref.shape[0]):
        xb = x_ref[k].astype(jnp.bfloat16)
        xb_ref[k] = xb
        p = _patches3(xb)
        y = jnp.dot(w_ref[...], p, preferred_element_type=jnp.float32)
        s = jnp.sum(y, axis=1, keepdims=True)
        ss = jnp.sum(y * y, axis=1, keepdims=True)
        c = jnp.concatenate([s, ss], axis=1)
        acc = c if acc is None else acc + c
    stats_ref[...] += acc


def _apply_kernel(xb_ref, w_ref, shift_ref, o_ref):
    for k in range(xb_ref.shape[0]):
        p = _patches3(xb_ref[k])
        y = jnp.dot(w_ref[...], p, preferred_element_type=jnp.float32)
        o_ref[k] = jnp.maximum(y + shift_ref[...], 0.0)


def kernel(x, conv_b, conv_w, bn_gamma, bn_beta):
    del conv_b
    n, c_in, l = x.shape
    c_out = conv_w.shape[0]
    m = n * l
    ck = 3 * c_in
    nb = _PASS1_BLOCK
    w_flat = conv_w.transpose(0, 2, 1).reshape(c_out, ck)

    params_seq = pltpu.CompilerParams(
        dimension_semantics=("arbitrary",), vmem_limit_bytes=_VMEM_LIMIT)

    xb, tot = pl.pallas_call(
        _stats_kernel,
        out_shape=(
            jax.ShapeDtypeStruct((n, c_in, l), jnp.bfloat16),
            jax.ShapeDtypeStruct((c_out, 2), jnp.float32),
        ),
        grid=(n // nb,),
        in_specs=[
            pl.BlockSpec((nb, c_in, l), lambda i: (i, 0, 0)),
            pl.BlockSpec((c_out, ck), lambda i: (0, 0)),
        ],
        out_specs=(
            pl.BlockSpec((nb, c_in, l), lambda i: (i, 0, 0)),
            pl.BlockSpec((c_out, 2), lambda i: (0, 0)),
        ),
        compiler_params=params_seq,
    )(x, w_flat.astype(jnp.bfloat16))

    mean = tot[:, 0] / m
    var = jnp.maximum(tot[:, 1] / m - mean * mean, 0.0)
    scale = bn_gamma * lax.rsqrt(var + _BN_EPS)
    shift = (bn_beta - mean * scale).reshape(c_out, 1)
    w_scaled = (w_flat * scale[:, None]).astype(jnp.bfloat16)

    nb2 = _PASS2_BLOCK
    return pl.pallas_call(
        _apply_kernel,
        out_shape=jax.ShapeDtypeStruct((n, c_out, l), jnp.float32),
        grid=(n // nb2,),
        in_specs=[
            pl.BlockSpec((nb2, c_in, l), lambda i: (i, 0, 0)),
            pl.BlockSpec((c_out, ck), lambda i: (0, 0)),
            pl.BlockSpec((c_out, 1), lambda i: (0, 0)),
        ],
        out_specs=pl.BlockSpec((nb2, c_out, l), lambda i: (i, 0, 0)),
        compiler_params=params_seq,
    )(xb, w_scaled, shift)
